# baseline (speedup 1.0000x reference)
import functools

import jax
import jax.numpy as jnp
from jax import lax
from jax.experimental import pallas as pl
from jax.experimental.pallas import tpu as pltpu
from jax.experimental.pallas import tpu_sc as plsc

_B, _N, _M = 8, 4096, 4096


_TILE_N = 512
_NT = _N // _TILE_N


def _tc_body(nb, x_ref, yt_ref, out_ref, min_yx, acc):
    b = pl.program_id(0)
    j = pl.program_id(1)

    @pl.when(jnp.logical_and(b == 0, j == 0))
    def _():
        out_ref[0, 0] = 0.0

    @pl.when(j == 0)
    def _():
        min_yx[...] = jnp.full((1, _M), jnp.inf, jnp.float32)
        acc[0] = 0.0
        acc[1] = 0.0

    x = x_ref[0]
    yt = yt_ref[0]

    mask = ((x[:, 0:1] != 10000.0)
            & (x[:, 1:2] != 10000.0)
            & (x[:, 2:3] != 10000.0))
    x0c = jnp.where(mask, x[:, 0:1], 1e19)

    d = jnp.zeros((_TILE_N, _M), jnp.float32)
    for k in range(3):
        xk = x0c if k == 0 else x[:, k:k + 1]
        diff = xk - yt[k:k + 1, :]
        d = d + diff * diff

    min_xy = jnp.min(d, axis=1, keepdims=True)
    acc[0] += jnp.sum(jnp.where(mask, min_xy, 0.0))
    acc[1] += jnp.sum(mask.astype(jnp.float32))

    part = jnp.min(d, axis=0, keepdims=True)
    min_yx[...] = jnp.minimum(min_yx[...], part)

    @pl.when(j == _NT - 1)
    def _():
        loss_b = acc[0] / acc[1] + jnp.sum(min_yx[...]) / _M
        out_ref[0, 0] += loss_b / _B


def _tc_part(fg, yt, nb):
    return pl.pallas_call(
        functools.partial(_tc_body, nb),
        grid=(nb, _NT),
        in_specs=[
            pl.BlockSpec((1, _TILE_N, 3), lambda b, j: (b, j, 0)),
            pl.BlockSpec((1, 3, _M), lambda b, j: (b, 0, 0)),
        ],
        out_specs=pl.BlockSpec(memory_space=pltpu.SMEM),
        out_shape=jax.ShapeDtypeStruct((1, 1), jnp.float32),
        scratch_shapes=[
            pltpu.VMEM((1, _M), jnp.float32),
            pltpu.SMEM((2,), jnp.float32),
        ],
    )(fg, yt)[0, 0]


_NB_SC = 2
_NB_TC = _B - _NB_SC
_NC, _NS, _L = 2, 16, 16
_BPC = _NB_SC // _NC
_TPB = _NS // _BPC
_RPT = _N // _TPB
_RI = 8
_SEG = _M // _TPB
_SENTINEL = 1e19


def _fold(v, op):
    lanes = lax.broadcasted_iota(jnp.int32, (_L,), 0)
    for k in (8, 4, 2, 1):
        v = op(v, v.at[lanes ^ k].get(mode='promise_in_bounds'))
    return v


def _sc_body(fg_hbm, prj_hbm, out_hbm,
             x0, x1, x2, y0, y1, y2, xraw, yraw, myx, tmp, vbuf,
             sh_myx, sh_sums, sh_sums2):
    c = lax.axis_index("c")
    s = lax.axis_index("s")
    b_local = s // _TPB
    q = s % _TPB
    b = _NB_TC + c * _BPC + b_local
    row0 = q * _RPT

    pltpu.sync_copy(fg_hbm.at[pl.ds((b * _N + row0) * 3, _RPT * 3)], xraw)
    pltpu.sync_copy(prj_hbm.at[pl.ds(b * _M * 3, _M * 3)], yraw)

    lanes_i = lax.broadcasted_iota(jnp.int32, (_L,), 0)

    def deint3(va, vb, vc, k):
        pos = lanes_i * 3 + k
        idx = pos & (_L - 1)
        ga = va.at[idx].get(mode='promise_in_bounds')
        gb = vb.at[idx].get(mode='promise_in_bounds')
        gc = vc.at[idx].get(mode='promise_in_bounds')
        return jnp.where(pos < _L, ga, jnp.where(pos < 2 * _L, gb, gc))

    def deint_y(i, _):
        va = yraw[pl.ds(i * 3 * _L, _L)]
        vb = yraw[pl.ds(i * 3 * _L + _L, _L)]
        vc = yraw[pl.ds(i * 3 * _L + 2 * _L, _L)]
        sl = pl.ds(i * _L, _L)
        y0[sl] = deint3(va, vb, vc, 0)
        y1[sl] = deint3(va, vb, vc, 1)
        y2[sl] = deint3(va, vb, vc, 2)
        return 0
    lax.fori_loop(0, _M // _L, deint_y, 0)

    def mask_step(i, cnt):
        va = xraw[pl.ds(i * 3 * _L, _L)]
        vb = xraw[pl.ds(i * 3 * _L + _L, _L)]
        vc = xraw[pl.ds(i * 3 * _L + 2 * _L, _L)]
        v0 = deint3(va, vb, vc, 0)
        v1 = deint3(va, vb, vc, 1)
        v2 = deint3(va, vb, vc, 2)
        m = (v0 != 10000.0) & (v1 != 10000.0) & (v2 != 10000.0)
        sl = pl.ds(i * _L, _L)
        x0[sl] = jnp.where(m, v0, _SENTINEL)
        x1[sl] = v1
        x2[sl] = v2
        return cnt + jnp.where(m, 1.0, 0.0)

    cnt_vec = lax.fori_loop(0, _RPT // _L, mask_step,
                            jnp.zeros((_L,), jnp.float32))
    count = _fold(cnt_vec, jnp.add)[0]

    def init_step(i, _):
        myx[pl.ds(i * _L, _L)] = jnp.full((_L,), jnp.inf, jnp.float32)
        return 0
    lax.fori_loop(0, _M // _L, init_step, 0)

    inf_vec = jnp.full((_L,), jnp.inf, jnp.float32)

    def row_block(ib, sum_xy):
        base = ib * _L
        xv0 = x0[pl.ds(base, _L)]
        xv1 = x1[pl.ds(base, _L)]
        xv2 = x2[pl.ds(base, _L)]
        xs0 = [(xv0[r], xv1[r], xv2[r]) for r in range(_RI)]
        xs1 = [(xv0[r], xv1[r], xv2[r]) for r in range(_RI, _L)]

        def half(xs, sum_xy):
            def y_step(j, minvecs):
                sl = pl.ds(j * _L, _L)
                yv0, yv1, yv2 = y0[sl], y1[sl], y2[sl]
                new_min = []
                bmin = None
                for r in range(_RI):
                    a = yv0 - xs[r][0]
                    d = a * a
                    a = yv1 - xs[r][1]
                    d = d + a * a
                    a = yv2 - xs[r][2]
                    d = d + a * a
                    new_min.append(jnp.minimum(minvecs[r], d))
                    bmin = d if bmin is None else jnp.minimum(bmin, d)
                myx[sl] = jnp.minimum(myx[sl], bmin)
                return tuple(new_min)

            minvecs = lax.fori_loop(0, _M // _L, y_step,
                                    tuple(inf_vec for _ in range(_RI)),
                                    unroll=2)
            for r in range(_RI):
                mn = _fold(minvecs[r], jnp.minimum)[0]
                valid = xs[r][0] < 1e18
                sum_xy = sum_xy + jnp.where(valid, mn, 0.0)
            return sum_xy

        return half(xs1, half(xs0, sum_xy))

    sum_xy = lax.fori_loop(0, _RPT // _L, row_block, jnp.float32(0.0))

    lanes = lax.broadcasted_iota(jnp.int32, (_L,), 0)
    packed = (jnp.where(lanes == 0, sum_xy, 0.0)
              + jnp.where(lanes == 1, count, 0.0))
    vbuf[...] = packed
    pltpu.sync_copy(myx, sh_myx.at[pl.ds(s * _M, _M)])
    pltpu.sync_copy(vbuf, sh_sums.at[pl.ds(s * _L, _L)])
    plsc.subcore_barrier()

    group0 = s - q
    for pt in range(_TPB):
        pltpu.sync_copy(
            sh_myx.at[pl.ds((group0 + pt) * _M + q * _SEG, _SEG)],
            tmp.at[pl.ds(pt * _SEG, _SEG)])

    def seg_step(i, ssum):
        acc = tmp[pl.ds(i * _L, _L)]
        for pt in range(1, _TPB):
            acc = jnp.minimum(acc, tmp[pl.ds(pt * _SEG + i * _L, _L)])
        return ssum + acc

    ssum = lax.fori_loop(0, _SEG // _L, seg_step,
                         jnp.zeros((_L,), jnp.float32))
    seg_total = _fold(ssum, jnp.add)
    vbuf[...] = jnp.where(lanes == 0, seg_total, 0.0)
    pltpu.sync_copy(vbuf, sh_sums2.at[pl.ds(s * _L, _L)])
    plsc.subcore_barrier()

    @pl.when(q == 0)
    def _():
        def lead_step(t, carry):
            svec, yxvec = carry
            pltpu.sync_copy(sh_sums.at[pl.ds((group0 + t) * _L, _L)], vbuf)
            svec = svec + vbuf[...]
            pltpu.sync_copy(sh_sums2.at[pl.ds((group0 + t) * _L, _L)], vbuf)
            yxvec = yxvec + vbuf[...]
            return svec, yxvec

        zero = jnp.zeros((_L,), jnp.float32)
        svec, yxvec = lax.fori_loop(0, _TPB, lead_step, (zero, zero))

        swapped = svec.at[lanes ^ 1].get(mode='promise_in_bounds')
        loss_xy = (svec / swapped)[0]
        loss_b = loss_xy + yxvec[0] * (1.0 / _M)
        vbuf[...] = jnp.zeros((_L,), jnp.float32) + loss_b
        pltpu.sync_copy(vbuf, out_hbm.at[pl.ds(b * _L, _L)])


def _make_sc_call():
    mesh = plsc.VectorSubcoreMesh(core_axis_name="c", subcore_axis_name="s",
                                  num_cores=_NC, num_subcores=_NS)
    return functools.partial(
        pl.kernel,
        out_type=jax.ShapeDtypeStruct((_NB_SC * _L,), jnp.float32),
        mesh=mesh,
        scratch_types=[
            pltpu.VMEM((_RPT,), jnp.float32),
            pltpu.VMEM((_RPT,), jnp.float32),
            pltpu.VMEM((_RPT,), jnp.float32),
            pltpu.VMEM((_M,), jnp.float32),
            pltpu.VMEM((_M,), jnp.float32),
            pltpu.VMEM((_M,), jnp.float32),
            pltpu.VMEM((_RPT * 3,), jnp.float32),
            pltpu.VMEM((_M * 3,), jnp.float32),
            pltpu.VMEM((_M,), jnp.float32),
            pltpu.VMEM((_M,), jnp.float32),
            pltpu.VMEM((_L,), jnp.float32),
            pltpu.VMEM_SHARED((_NS * _M,), jnp.float32),
            pltpu.VMEM_SHARED((_NS * _L,), jnp.float32),
            pltpu.VMEM_SHARED((_NS * _L,), jnp.float32),
        ],
    )(_sc_body)


@jax.jit
def _chamfer(fg, prj):
    sc_out = _make_sc_call()(fg.reshape(-1), prj.reshape(-1))
    total = jnp.sum(sc_out.reshape(_NB_SC, _L)[:, 0]) / _B
    if _NB_TC:
        yt = prj.transpose(0, 2, 1)
        total = total + _tc_part(fg, yt, _NB_TC)
    return total


def kernel(fg_points, prj_points, x_lengths):
    del x_lengths
    return _chamfer(fg_points.astype(jnp.float32),
                    prj_points.astype(jnp.float32))


# speedup vs baseline: 1.0005x; 1.0005x over previous
import functools

import jax
import jax.numpy as jnp
from jax import lax
from jax.experimental import pallas as pl
from jax.experimental.pallas import tpu as pltpu
from jax.experimental.pallas import tpu_sc as plsc

_B, _N, _M = 8, 4096, 4096


_TILE_N = 512
_NT = _N // _TILE_N


def _tc_body(nb, x_ref, yt_ref, out_ref, min_yx, acc):
    b = pl.program_id(0)
    j = pl.program_id(1)

    @pl.when(jnp.logical_and(b == 0, j == 0))
    def _():
        out_ref[0, 0] = 0.0

    @pl.when(j == 0)
    def _():
        min_yx[...] = jnp.full((1, _M), jnp.inf, jnp.float32)
        acc[0] = 0.0
        acc[1] = 0.0

    x = x_ref[0]
    yt = yt_ref[0]

    mask = ((x[:, 0:1] != 10000.0)
            & (x[:, 1:2] != 10000.0)
            & (x[:, 2:3] != 10000.0))
    x0c = jnp.where(mask, x[:, 0:1], 1e19)

    d = jnp.zeros((_TILE_N, _M), jnp.float32)
    for k in range(3):
        xk = x0c if k == 0 else x[:, k:k + 1]
        diff = xk - yt[k:k + 1, :]
        d = d + diff * diff

    min_xy = jnp.min(d, axis=1, keepdims=True)
    acc[0] += jnp.sum(jnp.where(mask, min_xy, 0.0))
    acc[1] += jnp.sum(mask.astype(jnp.float32))

    part = jnp.min(d, axis=0, keepdims=True)
    min_yx[...] = jnp.minimum(min_yx[...], part)

    @pl.when(j == _NT - 1)
    def _():
        loss_b = acc[0] / acc[1] + jnp.sum(min_yx[...]) / _M
        out_ref[0, 0] += loss_b / _B


def _tc_part(fg, yt, nb):
    return pl.pallas_call(
        functools.partial(_tc_body, nb),
        grid=(nb, _NT),
        in_specs=[
            pl.BlockSpec((1, _TILE_N, 3), lambda b, j: (b, j, 0)),
            pl.BlockSpec((1, 3, _M), lambda b, j: (b, 0, 0)),
        ],
        out_specs=pl.BlockSpec(memory_space=pltpu.SMEM),
        out_shape=jax.ShapeDtypeStruct((1, 1), jnp.float32),
        scratch_shapes=[
            pltpu.VMEM((1, _M), jnp.float32),
            pltpu.SMEM((2,), jnp.float32),
        ],
    )(fg, yt)[0, 0]


_NB_SC = 2
_NB_TC = _B - _NB_SC
_NC, _NS, _L = 2, 16, 16
_BPC = _NB_SC // _NC
_TPB = _NS // _BPC
_RPT = _N // _TPB
_RI = 8
_SEG = _M // _TPB
_SENTINEL = 1e19


def _fold(v, op):
    lanes = lax.broadcasted_iota(jnp.int32, (_L,), 0)
    for k in (8, 4, 2, 1):
        v = op(v, v.at[lanes ^ k].get(mode='promise_in_bounds'))
    return v


def _sc_body(fg_hbm, prj_hbm, out_hbm,
             x0, x1, x2, y0, y1, y2, xraw, yraw, myx, tmp, vbuf,
             sh_myx, sh_sums, sh_sums2):
    c = lax.axis_index("c")
    s = lax.axis_index("s")
    b_local = s // _TPB
    q = s % _TPB
    b_sc = c * _BPC + b_local
    b = _NB_TC + b_sc
    row0 = q * _RPT

    pltpu.sync_copy(fg_hbm.at[pl.ds((b * _N + row0) * 3, _RPT * 3)], xraw)
    pltpu.sync_copy(prj_hbm.at[pl.ds(b * _M * 3, _M * 3)], yraw)

    lanes_i = lax.broadcasted_iota(jnp.int32, (_L,), 0)

    def deint3(va, vb, vc, k):
        pos = lanes_i * 3 + k
        idx = pos & (_L - 1)
        ga = va.at[idx].get(mode='promise_in_bounds')
        gb = vb.at[idx].get(mode='promise_in_bounds')
        gc = vc.at[idx].get(mode='promise_in_bounds')
        return jnp.where(pos < _L, ga, jnp.where(pos < 2 * _L, gb, gc))

    def deint_y(i, _):
        va = yraw[pl.ds(i * 3 * _L, _L)]
        vb = yraw[pl.ds(i * 3 * _L + _L, _L)]
        vc = yraw[pl.ds(i * 3 * _L + 2 * _L, _L)]
        sl = pl.ds(i * _L, _L)
        y0[sl] = deint3(va, vb, vc, 0)
        y1[sl] = deint3(va, vb, vc, 1)
        y2[sl] = deint3(va, vb, vc, 2)
        return 0
    lax.fori_loop(0, _M // _L, deint_y, 0)

    def mask_step(i, cnt):
        va = xraw[pl.ds(i * 3 * _L, _L)]
        vb = xraw[pl.ds(i * 3 * _L + _L, _L)]
        vc = xraw[pl.ds(i * 3 * _L + 2 * _L, _L)]
        v0 = deint3(va, vb, vc, 0)
        v1 = deint3(va, vb, vc, 1)
        v2 = deint3(va, vb, vc, 2)
        m = (v0 != 10000.0) & (v1 != 10000.0) & (v2 != 10000.0)
        sl = pl.ds(i * _L, _L)
        x0[sl] = jnp.where(m, v0, _SENTINEL)
        x1[sl] = v1
        x2[sl] = v2
        return cnt + jnp.where(m, 1.0, 0.0)

    cnt_vec = lax.fori_loop(0, _RPT // _L, mask_step,
                            jnp.zeros((_L,), jnp.float32))
    count = _fold(cnt_vec, jnp.add)[0]

    def init_step(i, _):
        myx[pl.ds(i * _L, _L)] = jnp.full((_L,), jnp.inf, jnp.float32)
        return 0
    lax.fori_loop(0, _M // _L, init_step, 0)

    inf_vec = jnp.full((_L,), jnp.inf, jnp.float32)

    def row_block(ib, sum_xy):
        base = ib * _L
        xv0 = x0[pl.ds(base, _L)]
        xv1 = x1[pl.ds(base, _L)]
        xv2 = x2[pl.ds(base, _L)]
        xs0 = [(xv0[r], xv1[r], xv2[r]) for r in range(_RI)]
        xs1 = [(xv0[r], xv1[r], xv2[r]) for r in range(_RI, _L)]

        def half(xs, sum_xy):
            def y_step(j, minvecs):
                sl = pl.ds(j * _L, _L)
                yv0, yv1, yv2 = y0[sl], y1[sl], y2[sl]
                new_min = []
                bmin = None
                for r in range(_RI):
                    a = yv0 - xs[r][0]
                    d = a * a
                    a = yv1 - xs[r][1]
                    d = d + a * a
                    a = yv2 - xs[r][2]
                    d = d + a * a
                    new_min.append(jnp.minimum(minvecs[r], d))
                    bmin = d if bmin is None else jnp.minimum(bmin, d)
                myx[sl] = jnp.minimum(myx[sl], bmin)
                return tuple(new_min)

            minvecs = lax.fori_loop(0, _M // _L, y_step,
                                    tuple(inf_vec for _ in range(_RI)),
                                    unroll=2)
            for r in range(_RI):
                mn = _fold(minvecs[r], jnp.minimum)[0]
                valid = xs[r][0] < 1e18
                sum_xy = sum_xy + jnp.where(valid, mn, 0.0)
            return sum_xy

        return half(xs1, half(xs0, sum_xy))

    sum_xy = lax.fori_loop(0, _RPT // _L, row_block, jnp.float32(0.0))

    lanes = lax.broadcasted_iota(jnp.int32, (_L,), 0)
    packed = (jnp.where(lanes == 0, sum_xy, 0.0)
              + jnp.where(lanes == 1, count, 0.0))
    vbuf[...] = packed
    pltpu.sync_copy(myx, sh_myx.at[pl.ds(s * _M, _M)])
    pltpu.sync_copy(vbuf, sh_sums.at[pl.ds(s * _L, _L)])
    plsc.subcore_barrier()

    group0 = s - q
    for pt in range(_TPB):
        pltpu.sync_copy(
            sh_myx.at[pl.ds((group0 + pt) * _M + q * _SEG, _SEG)],
            tmp.at[pl.ds(pt * _SEG, _SEG)])

    def seg_step(i, ssum):
        acc = tmp[pl.ds(i * _L, _L)]
        for pt in range(1, _TPB):
            acc = jnp.minimum(acc, tmp[pl.ds(pt * _SEG + i * _L, _L)])
        return ssum + acc

    ssum = lax.fori_loop(0, _SEG // _L, seg_step,
                         jnp.zeros((_L,), jnp.float32))
    seg_total = _fold(ssum, jnp.add)
    vbuf[...] = jnp.where(lanes == 0, seg_total, 0.0)
    pltpu.sync_copy(vbuf, sh_sums2.at[pl.ds(s * _L, _L)])
    plsc.subcore_barrier()

    @pl.when(q == 0)
    def _():
        def lead_step(t, carry):
            svec, yxvec = carry
            pltpu.sync_copy(sh_sums.at[pl.ds((group0 + t) * _L, _L)], vbuf)
            svec = svec + vbuf[...]
            pltpu.sync_copy(sh_sums2.at[pl.ds((group0 + t) * _L, _L)], vbuf)
            yxvec = yxvec + vbuf[...]
            return svec, yxvec

        zero = jnp.zeros((_L,), jnp.float32)
        svec, yxvec = lax.fori_loop(0, _TPB, lead_step, (zero, zero))

        swapped = svec.at[lanes ^ 1].get(mode='promise_in_bounds')
        loss_xy = (svec / swapped)[0]
        loss_b = loss_xy + yxvec[0] * (1.0 / _M)
        vbuf[...] = jnp.zeros((_L,), jnp.float32) + loss_b
        pltpu.sync_copy(vbuf, out_hbm.at[pl.ds(b_sc * _L, _L)])


def _make_sc_call():
    mesh = plsc.VectorSubcoreMesh(core_axis_name="c", subcore_axis_name="s",
                                  num_cores=_NC, num_subcores=_NS)
    return functools.partial(
        pl.kernel,
        out_type=jax.ShapeDtypeStruct((_NB_SC * _L,), jnp.float32),
        mesh=mesh,
        scratch_types=[
            pltpu.VMEM((_RPT,), jnp.float32),
            pltpu.VMEM((_RPT,), jnp.float32),
            pltpu.VMEM((_RPT,), jnp.float32),
            pltpu.VMEM((_M,), jnp.float32),
            pltpu.VMEM((_M,), jnp.float32),
            pltpu.VMEM((_M,), jnp.float32),
            pltpu.VMEM((_RPT * 3,), jnp.float32),
            pltpu.VMEM((_M * 3,), jnp.float32),
            pltpu.VMEM((_M,), jnp.float32),
            pltpu.VMEM((_M,), jnp.float32),
            pltpu.VMEM((_L,), jnp.float32),
            pltpu.VMEM_SHARED((_NS * _M,), jnp.float32),
            pltpu.VMEM_SHARED((_NS * _L,), jnp.float32),
            pltpu.VMEM_SHARED((_NS * _L,), jnp.float32),
        ],
    )(_sc_body)


@jax.jit
def _chamfer(fg, prj):
    sc_out = _make_sc_call()(fg.reshape(-1), prj.reshape(-1))
    total = jnp.sum(sc_out.reshape(_NB_SC, _L)[:, 0]) / _B
    if _NB_TC:
        yt = prj.transpose(0, 2, 1)
        total = total + _tc_part(fg, yt, _NB_TC)
    return total


def kernel(fg_points, prj_points, x_lengths):
    del x_lengths
    return _chamfer(fg_points.astype(jnp.float32),
                    prj_points.astype(jnp.float32))


# speedup vs baseline: 1.1360x; 1.1355x over previous
import functools

import jax
import jax.numpy as jnp
from jax import lax
from jax.experimental import pallas as pl
from jax.experimental.pallas import tpu as pltpu
from jax.experimental.pallas import tpu_sc as plsc

_B, _N, _M = 8, 4096, 4096


_TILE_N = 512
_NT = _N // _TILE_N


def _tc_body(nb, x_ref, yt_ref, out_ref, min_yx, acc):
    b = pl.program_id(0)
    j = pl.program_id(1)

    @pl.when(jnp.logical_and(b == 0, j == 0))
    def _():
        out_ref[0, 0] = 0.0

    @pl.when(j == 0)
    def _():
        min_yx[...] = jnp.full((1, _M), jnp.inf, jnp.float32)
        acc[0] = 0.0
        acc[1] = 0.0

    x = x_ref[0]
    yt = yt_ref[0]

    mask = ((x[:, 0:1] != 10000.0)
            & (x[:, 1:2] != 10000.0)
            & (x[:, 2:3] != 10000.0))
    x0c = jnp.where(mask, x[:, 0:1], 1e19)

    d = jnp.zeros((_TILE_N, _M), jnp.float32)
    for k in range(3):
        xk = x0c if k == 0 else x[:, k:k + 1]
        diff = xk - yt[k:k + 1, :]
        d = d + diff * diff

    min_xy = jnp.min(d, axis=1, keepdims=True)
    acc[0] += jnp.sum(jnp.where(mask, min_xy, 0.0))
    acc[1] += jnp.sum(mask.astype(jnp.float32))

    part = jnp.min(d, axis=0, keepdims=True)
    min_yx[...] = jnp.minimum(min_yx[...], part)

    @pl.when(j == _NT - 1)
    def _():
        loss_b = acc[0] / acc[1] + jnp.sum(min_yx[...]) / _M
        out_ref[0, 0] += loss_b / _B


def _tc_part(fg, yt, nb):
    return pl.pallas_call(
        functools.partial(_tc_body, nb),
        grid=(nb, _NT),
        in_specs=[
            pl.BlockSpec((1, _TILE_N, 3), lambda b, j: (b, j, 0)),
            pl.BlockSpec((1, 3, _M), lambda b, j: (b, 0, 0)),
        ],
        out_specs=pl.BlockSpec(memory_space=pltpu.SMEM),
        out_shape=jax.ShapeDtypeStruct((1, 1), jnp.float32),
        scratch_shapes=[
            pltpu.VMEM((1, _M), jnp.float32),
            pltpu.SMEM((2,), jnp.float32),
        ],
    )(fg, yt)[0, 0]


_NB_SC = 2
_NB_TC = _B - _NB_SC
_NC, _NS, _L = 2, 16, 16
_BPC = _NB_SC // _NC
_TPB = _NS // _BPC
_RPT = _N // _TPB
_RI = 8
_SEG = _M // _TPB
_SENTINEL = 1e19


def _fold(v, op):
    lanes = lax.broadcasted_iota(jnp.int32, (_L,), 0)
    for k in (8, 4, 2, 1):
        v = op(v, v.at[lanes ^ k].get(mode='promise_in_bounds'))
    return v


def _sc_body(fg_hbm, prj_hbm, out_hbm,
             x0, x1, x2, y0, y1, y2, myx, tmp, vbuf,
             sh_myx, sh_sums, sh_sums2):
    c = lax.axis_index("c")
    s = lax.axis_index("s")
    b_local = s // _TPB
    q = s % _TPB
    b_sc = c * _BPC + b_local
    b = _NB_TC + b_sc
    row0 = q * _RPT

    xoff = (b_sc * 3) * _N + row0
    yoff = (b_sc * 3) * _M
    pltpu.sync_copy(fg_hbm.at[pl.ds(xoff, _RPT)], x0)
    pltpu.sync_copy(fg_hbm.at[pl.ds(xoff + _N, _RPT)], x1)
    pltpu.sync_copy(fg_hbm.at[pl.ds(xoff + 2 * _N, _RPT)], x2)
    pltpu.sync_copy(prj_hbm.at[pl.ds(yoff, _M)], y0)
    pltpu.sync_copy(prj_hbm.at[pl.ds(yoff + _M, _M)], y1)
    pltpu.sync_copy(prj_hbm.at[pl.ds(yoff + 2 * _M, _M)], y2)

    def mask_step(i, cnt):
        sl = pl.ds(i * _L, _L)
        v0, v1, v2 = x0[sl], x1[sl], x2[sl]
        m = (v0 != 10000.0) & (v1 != 10000.0) & (v2 != 10000.0)
        x0[sl] = jnp.where(m, v0, _SENTINEL)
        return cnt + jnp.where(m, 1.0, 0.0)

    cnt_vec = lax.fori_loop(0, _RPT // _L, mask_step,
                            jnp.zeros((_L,), jnp.float32))
    count = _fold(cnt_vec, jnp.add)[0]

    def init_step(i, _):
        myx[pl.ds(i * _L, _L)] = jnp.full((_L,), jnp.inf, jnp.float32)
        return 0
    lax.fori_loop(0, _M // _L, init_step, 0)

    inf_vec = jnp.full((_L,), jnp.inf, jnp.float32)

    def row_block(ib, sum_xy):
        base = ib * _L
        xv0 = x0[pl.ds(base, _L)]
        xv1 = x1[pl.ds(base, _L)]
        xv2 = x2[pl.ds(base, _L)]
        xs0 = [(xv0[r], xv1[r], xv2[r]) for r in range(_RI)]
        xs1 = [(xv0[r], xv1[r], xv2[r]) for r in range(_RI, _L)]

        def half(xs, sum_xy):
            def y_step(j, minvecs):
                sl = pl.ds(j * _L, _L)
                yv0, yv1, yv2 = y0[sl], y1[sl], y2[sl]
                new_min = []
                bmin = None
                for r in range(_RI):
                    a = yv0 - xs[r][0]
                    d = a * a
                    a = yv1 - xs[r][1]
                    d = d + a * a
                    a = yv2 - xs[r][2]
                    d = d + a * a
                    new_min.append(jnp.minimum(minvecs[r], d))
                    bmin = d if bmin is None else jnp.minimum(bmin, d)
                myx[sl] = jnp.minimum(myx[sl], bmin)
                return tuple(new_min)

            minvecs = lax.fori_loop(0, _M // _L, y_step,
                                    tuple(inf_vec for _ in range(_RI)),
                                    unroll=2)
            for r in range(_RI):
                mn = _fold(minvecs[r], jnp.minimum)[0]
                valid = xs[r][0] < 1e18
                sum_xy = sum_xy + jnp.where(valid, mn, 0.0)
            return sum_xy

        return half(xs1, half(xs0, sum_xy))

    sum_xy = lax.fori_loop(0, _RPT // _L, row_block, jnp.float32(0.0))

    lanes = lax.broadcasted_iota(jnp.int32, (_L,), 0)
    packed = (jnp.where(lanes == 0, sum_xy, 0.0)
              + jnp.where(lanes == 1, count, 0.0))
    vbuf[...] = packed
    pltpu.sync_copy(myx, sh_myx.at[pl.ds(s * _M, _M)])
    pltpu.sync_copy(vbuf, sh_sums.at[pl.ds(s * _L, _L)])
    plsc.subcore_barrier()

    group0 = s - q
    for pt in range(_TPB):
        pltpu.sync_copy(
            sh_myx.at[pl.ds((group0 + pt) * _M + q * _SEG, _SEG)],
            tmp.at[pl.ds(pt * _SEG, _SEG)])

    def seg_step(i, ssum):
        acc = tmp[pl.ds(i * _L, _L)]
        for pt in range(1, _TPB):
            acc = jnp.minimum(acc, tmp[pl.ds(pt * _SEG + i * _L, _L)])
        return ssum + acc

    ssum = lax.fori_loop(0, _SEG // _L, seg_step,
                         jnp.zeros((_L,), jnp.float32))
    seg_total = _fold(ssum, jnp.add)
    vbuf[...] = jnp.where(lanes == 0, seg_total, 0.0)
    pltpu.sync_copy(vbuf, sh_sums2.at[pl.ds(s * _L, _L)])
    plsc.subcore_barrier()

    @pl.when(q == 0)
    def _():
        def lead_step(t, carry):
            svec, yxvec = carry
            pltpu.sync_copy(sh_sums.at[pl.ds((group0 + t) * _L, _L)], vbuf)
            svec = svec + vbuf[...]
            pltpu.sync_copy(sh_sums2.at[pl.ds((group0 + t) * _L, _L)], vbuf)
            yxvec = yxvec + vbuf[...]
            return svec, yxvec

        zero = jnp.zeros((_L,), jnp.float32)
        svec, yxvec = lax.fori_loop(0, _TPB, lead_step, (zero, zero))

        swapped = svec.at[lanes ^ 1].get(mode='promise_in_bounds')
        loss_xy = (svec / swapped)[0]
        loss_b = loss_xy + yxvec[0] * (1.0 / _M)
        vbuf[...] = jnp.zeros((_L,), jnp.float32) + loss_b
        pltpu.sync_copy(vbuf, out_hbm.at[pl.ds(b_sc * _L, _L)])


def _make_sc_call():
    mesh = plsc.VectorSubcoreMesh(core_axis_name="c", subcore_axis_name="s",
                                  num_cores=_NC, num_subcores=_NS)
    return functools.partial(
        pl.kernel,
        out_type=jax.ShapeDtypeStruct((_NB_SC * _L,), jnp.float32),
        mesh=mesh,
        scratch_types=[
            pltpu.VMEM((_RPT,), jnp.float32),
            pltpu.VMEM((_RPT,), jnp.float32),
            pltpu.VMEM((_RPT,), jnp.float32),
            pltpu.VMEM((_M,), jnp.float32),
            pltpu.VMEM((_M,), jnp.float32),
            pltpu.VMEM((_M,), jnp.float32),
            pltpu.VMEM((_M,), jnp.float32),
            pltpu.VMEM((_M,), jnp.float32),
            pltpu.VMEM((_L,), jnp.float32),
            pltpu.VMEM_SHARED((_NS * _M,), jnp.float32),
            pltpu.VMEM_SHARED((_NS * _L,), jnp.float32),
            pltpu.VMEM_SHARED((_NS * _L,), jnp.float32),
        ],
    )(_sc_body)


@jax.jit
def _chamfer(fg, prj):
    fgt = fg[_NB_TC:].transpose(0, 2, 1)
    prjt = prj[_NB_TC:].transpose(0, 2, 1)
    sc_out = _make_sc_call()(fgt.reshape(-1), prjt.reshape(-1))
    total = jnp.sum(sc_out.reshape(_NB_SC, _L)[:, 0]) / _B
    if _NB_TC:
        yt = prj.transpose(0, 2, 1)
        total = total + _tc_part(fg, yt, _NB_TC)
    return total


def kernel(fg_points, prj_points, x_lengths):
    del x_lengths
    return _chamfer(fg_points.astype(jnp.float32),
                    prj_points.astype(jnp.float32))


# speedup vs baseline: 1.2447x; 1.0957x over previous
import functools

import jax
import jax.numpy as jnp
from jax import lax
from jax.experimental import pallas as pl
from jax.experimental.pallas import tpu as pltpu
from jax.experimental.pallas import tpu_sc as plsc

_B, _N, _M = 8, 4096, 4096


_TILE_N = 512
_NT = _N // _TILE_N


def _tc_body(nb, x_ref, yt_ref, out_ref, min_yx, acc):
    b = pl.program_id(0)
    j = pl.program_id(1)

    @pl.when(jnp.logical_and(b == 0, j == 0))
    def _():
        out_ref[0, 0] = 0.0

    @pl.when(j == 0)
    def _():
        min_yx[...] = jnp.full((1, _M), jnp.inf, jnp.float32)
        acc[0] = 0.0
        acc[1] = 0.0

    x = x_ref[0]
    yt = yt_ref[0]

    mask = ((x[:, 0:1] != 10000.0)
            & (x[:, 1:2] != 10000.0)
            & (x[:, 2:3] != 10000.0))
    xs = jnp.where(mask, x, 1e19)

    xy = jax.lax.dot_general(xs, yt, (((1,), (0,)), ((), ())),
                             preferred_element_type=jnp.float32)
    x2 = jnp.sum(xs * xs, axis=1, keepdims=True)
    y2 = jnp.sum(yt * yt, axis=0, keepdims=True)
    d = (x2 + y2) - 2.0 * xy

    min_xy = jnp.min(d, axis=1, keepdims=True)
    acc[0] += jnp.sum(jnp.where(mask, min_xy, 0.0))
    acc[1] += jnp.sum(mask.astype(jnp.float32))

    part = jnp.min(d, axis=0, keepdims=True)
    min_yx[...] = jnp.minimum(min_yx[...], part)

    @pl.when(j == _NT - 1)
    def _():
        loss_b = acc[0] / acc[1] + jnp.sum(min_yx[...]) / _M
        out_ref[0, 0] += loss_b / _B


def _tc_part(fg, yt, nb):
    return pl.pallas_call(
        functools.partial(_tc_body, nb),
        grid=(nb, _NT),
        in_specs=[
            pl.BlockSpec((1, _TILE_N, 3), lambda b, j: (b, j, 0)),
            pl.BlockSpec((1, 3, _M), lambda b, j: (b, 0, 0)),
        ],
        out_specs=pl.BlockSpec(memory_space=pltpu.SMEM),
        out_shape=jax.ShapeDtypeStruct((1, 1), jnp.float32),
        scratch_shapes=[
            pltpu.VMEM((1, _M), jnp.float32),
            pltpu.SMEM((2,), jnp.float32),
        ],
    )(fg, yt)[0, 0]


_NB_SC = 2
_NB_TC = _B - _NB_SC
_NC, _NS, _L = 2, 16, 16
_BPC = _NB_SC // _NC
_TPB = _NS // _BPC
_RPT = _N // _TPB
_RI = 8
_SEG = _M // _TPB
_SENTINEL = 1e19


def _fold(v, op):
    lanes = lax.broadcasted_iota(jnp.int32, (_L,), 0)
    for k in (8, 4, 2, 1):
        v = op(v, v.at[lanes ^ k].get(mode='promise_in_bounds'))
    return v


def _sc_body(fg_hbm, prj_hbm, out_hbm,
             x0, x1, x2, y0, y1, y2, myx, tmp, vbuf,
             sh_myx, sh_sums, sh_sums2):
    c = lax.axis_index("c")
    s = lax.axis_index("s")
    b_local = s // _TPB
    q = s % _TPB
    b_sc = c * _BPC + b_local
    b = _NB_TC + b_sc
    row0 = q * _RPT

    xoff = (b_sc * 3) * _N + row0
    yoff = (b_sc * 3) * _M
    pltpu.sync_copy(fg_hbm.at[pl.ds(xoff, _RPT)], x0)
    pltpu.sync_copy(fg_hbm.at[pl.ds(xoff + _N, _RPT)], x1)
    pltpu.sync_copy(fg_hbm.at[pl.ds(xoff + 2 * _N, _RPT)], x2)
    pltpu.sync_copy(prj_hbm.at[pl.ds(yoff, _M)], y0)
    pltpu.sync_copy(prj_hbm.at[pl.ds(yoff + _M, _M)], y1)
    pltpu.sync_copy(prj_hbm.at[pl.ds(yoff + 2 * _M, _M)], y2)

    def mask_step(i, cnt):
        sl = pl.ds(i * _L, _L)
        v0, v1, v2 = x0[sl], x1[sl], x2[sl]
        m = (v0 != 10000.0) & (v1 != 10000.0) & (v2 != 10000.0)
        x0[sl] = jnp.where(m, v0, _SENTINEL)
        return cnt + jnp.where(m, 1.0, 0.0)

    cnt_vec = lax.fori_loop(0, _RPT // _L, mask_step,
                            jnp.zeros((_L,), jnp.float32))
    count = _fold(cnt_vec, jnp.add)[0]

    def init_step(i, _):
        myx[pl.ds(i * _L, _L)] = jnp.full((_L,), jnp.inf, jnp.float32)
        return 0
    lax.fori_loop(0, _M // _L, init_step, 0)

    inf_vec = jnp.full((_L,), jnp.inf, jnp.float32)

    def row_block(ib, sum_xy):
        base = ib * _L
        xv0 = x0[pl.ds(base, _L)]
        xv1 = x1[pl.ds(base, _L)]
        xv2 = x2[pl.ds(base, _L)]
        xs0 = [(xv0[r], xv1[r], xv2[r]) for r in range(_RI)]
        xs1 = [(xv0[r], xv1[r], xv2[r]) for r in range(_RI, _L)]

        def half(xs, sum_xy):
            def y_step(j, minvecs):
                sl = pl.ds(j * _L, _L)
                yv0, yv1, yv2 = y0[sl], y1[sl], y2[sl]
                new_min = []
                bmin = None
                for r in range(_RI):
                    a = yv0 - xs[r][0]
                    d = a * a
                    a = yv1 - xs[r][1]
                    d = d + a * a
                    a = yv2 - xs[r][2]
                    d = d + a * a
                    new_min.append(jnp.minimum(minvecs[r], d))
                    bmin = d if bmin is None else jnp.minimum(bmin, d)
                myx[sl] = jnp.minimum(myx[sl], bmin)
                return tuple(new_min)

            minvecs = lax.fori_loop(0, _M // _L, y_step,
                                    tuple(inf_vec for _ in range(_RI)),
                                    unroll=2)
            for r in range(_RI):
                mn = _fold(minvecs[r], jnp.minimum)[0]
                valid = xs[r][0] < 1e18
                sum_xy = sum_xy + jnp.where(valid, mn, 0.0)
            return sum_xy

        return half(xs1, half(xs0, sum_xy))

    sum_xy = lax.fori_loop(0, _RPT // _L, row_block, jnp.float32(0.0))

    lanes = lax.broadcasted_iota(jnp.int32, (_L,), 0)
    packed = (jnp.where(lanes == 0, sum_xy, 0.0)
              + jnp.where(lanes == 1, count, 0.0))
    vbuf[...] = packed
    pltpu.sync_copy(myx, sh_myx.at[pl.ds(s * _M, _M)])
    pltpu.sync_copy(vbuf, sh_sums.at[pl.ds(s * _L, _L)])
    plsc.subcore_barrier()

    group0 = s - q
    for pt in range(_TPB):
        pltpu.sync_copy(
            sh_myx.at[pl.ds((group0 + pt) * _M + q * _SEG, _SEG)],
            tmp.at[pl.ds(pt * _SEG, _SEG)])

    def seg_step(i, ssum):
        acc = tmp[pl.ds(i * _L, _L)]
        for pt in range(1, _TPB):
            acc = jnp.minimum(acc, tmp[pl.ds(pt * _SEG + i * _L, _L)])
        return ssum + acc

    ssum = lax.fori_loop(0, _SEG // _L, seg_step,
                         jnp.zeros((_L,), jnp.float32))
    seg_total = _fold(ssum, jnp.add)
    vbuf[...] = jnp.where(lanes == 0, seg_total, 0.0)
    pltpu.sync_copy(vbuf, sh_sums2.at[pl.ds(s * _L, _L)])
    plsc.subcore_barrier()

    @pl.when(q == 0)
    def _():
        def lead_step(t, carry):
            svec, yxvec = carry
            pltpu.sync_copy(sh_sums.at[pl.ds((group0 + t) * _L, _L)], vbuf)
            svec = svec + vbuf[...]
            pltpu.sync_copy(sh_sums2.at[pl.ds((group0 + t) * _L, _L)], vbuf)
            yxvec = yxvec + vbuf[...]
            return svec, yxvec

        zero = jnp.zeros((_L,), jnp.float32)
        svec, yxvec = lax.fori_loop(0, _TPB, lead_step, (zero, zero))

        swapped = svec.at[lanes ^ 1].get(mode='promise_in_bounds')
        loss_xy = (svec / swapped)[0]
        loss_b = loss_xy + yxvec[0] * (1.0 / _M)
        vbuf[...] = jnp.zeros((_L,), jnp.float32) + loss_b
        pltpu.sync_copy(vbuf, out_hbm.at[pl.ds(b_sc * _L, _L)])


def _make_sc_call():
    mesh = plsc.VectorSubcoreMesh(core_axis_name="c", subcore_axis_name="s",
                                  num_cores=_NC, num_subcores=_NS)
    return functools.partial(
        pl.kernel,
        out_type=jax.ShapeDtypeStruct((_NB_SC * _L,), jnp.float32),
        mesh=mesh,
        scratch_types=[
            pltpu.VMEM((_RPT,), jnp.float32),
            pltpu.VMEM((_RPT,), jnp.float32),
            pltpu.VMEM((_RPT,), jnp.float32),
            pltpu.VMEM((_M,), jnp.float32),
            pltpu.VMEM((_M,), jnp.float32),
            pltpu.VMEM((_M,), jnp.float32),
            pltpu.VMEM((_M,), jnp.float32),
            pltpu.VMEM((_M,), jnp.float32),
            pltpu.VMEM((_L,), jnp.float32),
            pltpu.VMEM_SHARED((_NS * _M,), jnp.float32),
            pltpu.VMEM_SHARED((_NS * _L,), jnp.float32),
            pltpu.VMEM_SHARED((_NS * _L,), jnp.float32),
        ],
    )(_sc_body)


@jax.jit
def _chamfer(fg, prj):
    fgt = fg[_NB_TC:].transpose(0, 2, 1)
    prjt = prj[_NB_TC:].transpose(0, 2, 1)
    sc_out = _make_sc_call()(fgt.reshape(-1), prjt.reshape(-1))
    total = jnp.sum(sc_out.reshape(_NB_SC, _L)[:, 0]) / _B
    if _NB_TC:
        yt = prj.transpose(0, 2, 1)
        total = total + _tc_part(fg, yt, _NB_TC)
    return total


def kernel(fg_points, prj_points, x_lengths):
    del x_lengths
    return _chamfer(fg_points.astype(jnp.float32),
                    prj_points.astype(jnp.float32))
